# Initial kernel scaffold; baseline (speedup 1.0000x reference)
import jax
import jax.numpy as jnp
from jax.experimental import pallas as pl


def kernel(leaf):
    raise NotImplementedError("write your pallas kernel here")


# speedup vs baseline: 7.9523x; 7.9523x over previous
import jax
import jax.numpy as jnp
from jax.experimental import pallas as pl

_N = 1000000
_BLK = 4096


def _keys_body(leaf_t_ref, k_ref):
    x = leaf_t_ref[0, :].astype(jnp.int32)
    y = leaf_t_ref[1, :].astype(jnp.int32)
    z = leaf_t_ref[2, :].astype(jnp.int32)
    k_ref[0, :] = (
        ((x >> 1) << 19) | ((y >> 1) << 11) | ((z >> 1) << 3)
        | ((x & 1) << 2) | ((y & 1) << 1) | (z & 1)
    )


def _compute_keys(leaf):
    n = leaf.shape[0]
    pad = (-n) % _BLK
    leaf_t = jnp.pad(leaf.T, ((0, 0), (0, pad)), constant_values=511.0)
    npad = n + pad
    k = pl.pallas_call(
        _keys_body,
        grid=(npad // _BLK,),
        in_specs=[pl.BlockSpec((3, _BLK), lambda i: (0, i))],
        out_specs=pl.BlockSpec((1, _BLK), lambda i: (0, i)),
        out_shape=jax.ShapeDtypeStruct((1, npad), jnp.int32),
    )(leaf_t)
    return k[0, :n]


def kernel(leaf):
    n = leaf.shape[0]
    k = _compute_keys(leaf)
    ks = jnp.sort(k)
    pk = ks >> 3
    oi = ks & 7
    first = jnp.ones((n,), dtype=jnp.bool_)
    newp = first.at[1:].set(pk[1:] != pk[:-1])
    newk = first.at[1:].set(ks[1:] != ks[:-1])
    rank = jnp.cumsum(newp.astype(jnp.int32)) - 1

    px = pk >> 16
    py = (pk >> 8) & 255
    pz = pk & 255
    coords = jnp.stack([px, py, pz], axis=1)
    pidx = jnp.where(newp, rank, n)
    parent_C = jnp.full((n, 3), -1, dtype=jnp.int32).at[pidx].set(
        coords, mode="drop")
    occupancy = jnp.zeros((n, 8), dtype=jnp.float32).at[rank, oi].add(
        newk.astype(jnp.float32), mode="drop")
    return parent_C, occupancy
